# Initial kernel scaffold; baseline (speedup 1.0000x reference)
import functools

import jax
import jax.numpy as jnp
from jax import lax
from jax.experimental import pallas as pl
from jax.experimental.pallas import tpu as pltpu
from jax.experimental.pallas import tpu_sc as plsc

N_PAD = 1024
NODES_PER_SC = 64
E_PAD = 256
IN_D = 128
OUT_D = 16
L = 16


def _sc_body(x_hbm, wt_hbm, bias_hbm, src_hbm, dst_hbm, out_hbm,
             x_v, wt_v, bias_v, src_v, dst_v, deg_v, dis_v, disall_v,
             h_v, o_v, hrows_v, msgs_v, sidx_v, didx_v, norm_v,
             h_sh, dis_sh, out_sh):
    s = lax.axis_index("s")
    base = s * NODES_PER_SC

    pltpu.sync_copy(x_hbm.at[pl.ds(base, NODES_PER_SC), :], x_v)
    pltpu.sync_copy(wt_hbm, wt_v)
    pltpu.sync_copy(bias_hbm, bias_v)
    pltpu.sync_copy(src_hbm, src_v)
    pltpu.sync_copy(dst_hbm, dst_v)

    ones = jnp.full((L,), 1.0, jnp.float32)
    for i in range(NODES_PER_SC // L):
        deg_v[pl.ds(i * L, L)] = ones

    for g in range(E_PAD // L):
        sv = src_v[pl.ds(g * L, L)]
        dv = dst_v[pl.ds(g * L, L)]
        keep = (sv != dv) & (dv >= base) & (dv < base + NODES_PER_SC)
        plsc.addupdate_scatter(deg_v, [dv - base], ones, mask=keep)

    for i in range(NODES_PER_SC // L):
        d = deg_v[pl.ds(i * L, L)]
        y = plsc.bitcast(
            jnp.int32(0x5F3759DF) - (plsc.bitcast(d, jnp.int32) >> 1),
            jnp.float32)
        for _ in range(3):
            y = y * (1.5 - 0.5 * d * y * y)
        dis_v[pl.ds(i * L, L)] = y

    bias_vec = bias_v[...]

    def mm_body(t, carry):
        n0 = t * 4
        accs = [jnp.zeros((L,), jnp.float32) for _ in range(4)]
        for k in range(IN_D):
            wrow = wt_v[k, :]
            for u in range(4):
                accs[u] = accs[u] + x_v[n0 + u, k] * wrow
        for u in range(4):
            h_v[n0 + u, :] = accs[u]
            dd = dis_v[n0 + u]
            o_v[n0 + u, :] = accs[u] * (dd * dd) + bias_vec
        return carry

    lax.fori_loop(0, NODES_PER_SC // 4, mm_body, 0)

    pltpu.sync_copy(h_v, h_sh.at[pl.ds(base, NODES_PER_SC), :])
    pltpu.sync_copy(dis_v, dis_sh.at[pl.ds(base, NODES_PER_SC)])
    pltpu.sync_copy(o_v, out_sh.at[pl.ds(base, NODES_PER_SC), :])
    plsc.subcore_barrier()

    pltpu.sync_copy(dis_sh, disall_v)
    sv = src_v[pl.ds(s * L, L)]
    dv = dst_v[pl.ds(s * L, L)]
    ew = jnp.where(sv != dv, 1.0, 0.0).astype(jnp.float32)
    dis_s = plsc.load_gather(disall_v, [sv])
    dis_d = plsc.load_gather(disall_v, [dv])
    norm_v[...] = dis_s * ew * dis_d
    sidx_v[...] = sv
    didx_v[...] = dv
    pltpu.sync_copy(h_sh.at[sidx_v], hrows_v)
    for j in range(L):
        msgs_v[j, :] = hrows_v[j, :] * norm_v[j]
    pltpu.sync_copy(msgs_v, out_sh.at[didx_v], add=True)
    plsc.subcore_barrier()

    pltpu.sync_copy(out_sh.at[pl.ds(base, NODES_PER_SC), :], o_v)
    pltpu.sync_copy(o_v, out_hbm.at[pl.ds(base, NODES_PER_SC), :])


@jax.jit
def _run(x_pad, wt, bias, src_pad, dst_pad):
    mesh = plsc.VectorSubcoreMesh(
        core_axis_name="c", subcore_axis_name="s", num_cores=1,
        num_subcores=16)
    f = pl.kernel(
        _sc_body,
        out_type=jax.ShapeDtypeStruct((N_PAD, OUT_D), jnp.float32),
        mesh=mesh,
        scratch_types=[
            pltpu.VMEM((NODES_PER_SC, IN_D), jnp.float32),
            pltpu.VMEM((IN_D, OUT_D), jnp.float32),
            pltpu.VMEM((OUT_D,), jnp.float32),
            pltpu.VMEM((E_PAD,), jnp.int32),
            pltpu.VMEM((E_PAD,), jnp.int32),
            pltpu.VMEM((NODES_PER_SC,), jnp.float32),
            pltpu.VMEM((NODES_PER_SC,), jnp.float32),
            pltpu.VMEM((N_PAD,), jnp.float32),
            pltpu.VMEM((NODES_PER_SC, OUT_D), jnp.float32),
            pltpu.VMEM((NODES_PER_SC, OUT_D), jnp.float32),
            pltpu.VMEM((L, OUT_D), jnp.float32),
            pltpu.VMEM((L, OUT_D), jnp.float32),
            pltpu.VMEM((L,), jnp.int32),
            pltpu.VMEM((L,), jnp.int32),
            pltpu.VMEM((L,), jnp.float32),
            pltpu.VMEM_SHARED((N_PAD, OUT_D), jnp.float32),
            pltpu.VMEM_SHARED((N_PAD,), jnp.float32),
            pltpu.VMEM_SHARED((N_PAD, OUT_D), jnp.float32),
        ],
    )
    return f(x_pad, wt, bias, src_pad, dst_pad)


def kernel(L_args_0_, L_args_1_,
           L_self_modules_encoder_modules_lin_parameters_weight_,
           L_self_modules_encoder_parameters_bias_):
    x = L_args_0_
    edge_index = L_args_1_.astype(jnp.int32)
    weight = L_self_modules_encoder_modules_lin_parameters_weight_
    bias = L_self_modules_encoder_parameters_bias_

    n_nodes, _ = x.shape
    n_edges = edge_index.shape[1]
    x_pad = jnp.pad(x, ((0, N_PAD - n_nodes), (0, 0)))
    src_pad = jnp.pad(edge_index[0], (0, E_PAD - n_edges))
    dst_pad = jnp.pad(edge_index[1], (0, E_PAD - n_edges))
    wt = weight.T

    out = _run(x_pad, wt, bias, src_pad, dst_pad)
    return (out[:n_nodes],)


# speedup vs baseline: 2.0067x; 2.0067x over previous
import jax
import jax.numpy as jnp
from jax import lax
from jax.experimental import pallas as pl
from jax.experimental.pallas import tpu as pltpu
from jax.experimental.pallas import tpu_sc as plsc

N_PAD = 1024
NPS = 64
E_PAD = 256
IN_D = 128
OUT_D = 16
L = 16


def _sc_body(x_hbm, wt_hbm, bias_hbm, src_hbm, dst_hbm,
             out_hbm, h_hbm, dis_hbm,
             x_v, wt_v, bias_v, src_v, dst_v,
             deg_v, dis_v, disall_v, h8_v, o_v, hall_v):
    s = lax.axis_index("s")
    base = s * NPS

    pltpu.sync_copy(x_hbm.at[pl.ds(base, NPS), :], x_v)
    pltpu.sync_copy(wt_hbm, wt_v)
    pltpu.sync_copy(bias_hbm, bias_v)
    pltpu.sync_copy(src_hbm, src_v)
    pltpu.sync_copy(dst_hbm, dst_v)

    ones = jnp.full((L,), 1.0, jnp.float32)
    for i in range(NPS // L):
        deg_v[pl.ds(i * L, L)] = ones

    for g in range(E_PAD // L):
        sv = src_v[pl.ds(g * L, L)]
        dv = dst_v[pl.ds(g * L, L)]
        keep = (sv != dv) & (dv >= base) & (dv < base + NPS)
        plsc.addupdate_scatter(deg_v, [dv - base], ones, mask=keep)

    for i in range(NPS // L):
        d = deg_v[pl.ds(i * L, L)]
        y = plsc.bitcast(
            jnp.int32(0x5F3759DF) - (plsc.bitcast(d, jnp.int32) >> 1),
            jnp.float32)
        for _ in range(3):
            y = y * (1.5 - 0.5 * d * y * y)
        dis_v[pl.ds(i * L, L)] = y

    bias_vec = bias_v[...]

    def mm_body(t, carry):
        n0 = t * 4
        accs = [jnp.zeros((L,), jnp.float32) for _ in range(4)]
        for kb in range(IN_D // L):
            xvs = [x_v[n0 + u, pl.ds(kb * L, L)] for u in range(4)]
            for j in range(L):
                wrow = wt_v[kb * L + j, :]
                for u in range(4):
                    accs[u] = accs[u] + xvs[u][j] * wrow
        for u in range(4):
            n = n0 + u
            h8_v[n >> 3, pl.ds((n & 7) * L, L)] = accs[u]
        return carry

    lax.fori_loop(0, NPS // 4, mm_body, 0)

    def scale_body(t, carry):
        n0 = t * L
        dvec = dis_v[pl.ds(n0, L)]
        dsq = dvec * dvec
        for u in range(L):
            n = n0 + u
            hrow = h8_v[n >> 3, pl.ds((n & 7) * L, L)]
            o_v[n, :] = hrow * dsq[u] + bias_vec
        return carry

    lax.fori_loop(0, NPS // L, scale_body, 0)

    pltpu.sync_copy(h8_v, h_hbm.at[pl.ds(s * (NPS // 8), NPS // 8), :])
    pltpu.sync_copy(dis_v, dis_hbm.at[pl.ds(base, NPS)])
    plsc.subcore_barrier()

    pltpu.sync_copy(dis_hbm, disall_v)
    pltpu.sync_copy(h_hbm, hall_v)

    def edge_body(g, carry):
        e0 = g * L
        sv = src_v[pl.ds(e0, L)]
        dv = dst_v[pl.ds(e0, L)]
        ew = jnp.where(sv != dv, 1.0, 0.0).astype(jnp.float32)
        dis_s = plsc.load_gather(disall_v, [sv])
        dis_d = plsc.load_gather(disall_v, [dv])
        nv = dis_s * ew * dis_d
        ldv = dv - base
        for j in range(L):
            lj = ldv[j]
            nj = nv[j]
            sj = sv[j]

            @pl.when((lj >= 0) & (lj < NPS))
            def _():
                hrow = hall_v[sj >> 3, pl.ds((sj & 7) * L, L)]
                o_v[lj, :] = o_v[lj, :] + hrow * nj

        return carry

    lax.fori_loop(0, E_PAD // L, edge_body, 0)

    pltpu.sync_copy(o_v, out_hbm.at[pl.ds(base, NPS), :])


@jax.jit
def _run(x_pad, wt, bias, src_pad, dst_pad):
    mesh = plsc.VectorSubcoreMesh(
        core_axis_name="c", subcore_axis_name="s", num_cores=1,
        num_subcores=16)
    f = pl.kernel(
        _sc_body,
        out_type=(jax.ShapeDtypeStruct((N_PAD, OUT_D), jnp.float32),
                  jax.ShapeDtypeStruct((N_PAD // 8, 8 * OUT_D), jnp.float32),
                  jax.ShapeDtypeStruct((N_PAD,), jnp.float32)),
        mesh=mesh,
        scratch_types=[
            pltpu.VMEM((NPS, IN_D), jnp.float32),
            pltpu.VMEM((IN_D, OUT_D), jnp.float32),
            pltpu.VMEM((OUT_D,), jnp.float32),
            pltpu.VMEM((E_PAD,), jnp.int32),
            pltpu.VMEM((E_PAD,), jnp.int32),
            pltpu.VMEM((NPS,), jnp.float32),
            pltpu.VMEM((NPS,), jnp.float32),
            pltpu.VMEM((N_PAD,), jnp.float32),
            pltpu.VMEM((NPS // 8, 8 * OUT_D), jnp.float32),
            pltpu.VMEM((NPS, OUT_D), jnp.float32),
            pltpu.VMEM((N_PAD // 8, 8 * OUT_D), jnp.float32),
        ],
        compiler_params=pltpu.CompilerParams(needs_layout_passes=False),
    )
    return f(x_pad, wt, bias, src_pad, dst_pad)


def kernel(L_args_0_, L_args_1_,
           L_self_modules_encoder_modules_lin_parameters_weight_,
           L_self_modules_encoder_parameters_bias_):
    x = L_args_0_
    edge_index = L_args_1_.astype(jnp.int32)
    weight = L_self_modules_encoder_modules_lin_parameters_weight_
    bias = L_self_modules_encoder_parameters_bias_

    n_nodes, _ = x.shape
    n_edges = edge_index.shape[1]
    x_pad = jnp.pad(x, ((0, N_PAD - n_nodes), (0, 0)))
    src_pad = jnp.pad(edge_index[0], (0, E_PAD - n_edges))
    dst_pad = jnp.pad(edge_index[1], (0, E_PAD - n_edges))
    wt = weight.T

    out, _, _ = _run(x_pad, wt, bias, src_pad, dst_pad)
    return (out[:n_nodes],)


# speedup vs baseline: 2.0965x; 1.0448x over previous
import jax
import jax.numpy as jnp
from jax import lax
from jax.experimental import pallas as pl
from jax.experimental.pallas import tpu as pltpu
from jax.experimental.pallas import tpu_sc as plsc

N_PAD = 1024
NPS = 64
E_PAD = 112
IN_D = 128
OUT_D = 16
L = 16


def _sc_body(x_hbm, wt_hbm, bias_hbm, src_hbm, dst_hbm,
             out_hbm, h_hbm, dis_hbm,
             x_v, wt_v, bias_v, src_v, dst_v,
             deg_v, dis_v, disall_v, h8_v, o_v, hall_v):
    s = lax.axis_index("s")
    base = s * NPS

    pltpu.sync_copy(x_hbm.at[pl.ds(base, NPS), :], x_v)
    pltpu.sync_copy(wt_hbm, wt_v)
    pltpu.sync_copy(bias_hbm, bias_v)
    pltpu.sync_copy(src_hbm, src_v)
    pltpu.sync_copy(dst_hbm, dst_v)

    ones = jnp.full((L,), 1.0, jnp.float32)
    for i in range(NPS // L):
        deg_v[pl.ds(i * L, L)] = ones

    for g in range(E_PAD // L):
        sv = src_v[pl.ds(g * L, L)]
        dv = dst_v[pl.ds(g * L, L)]
        keep = (sv != dv) & (dv >= base) & (dv < base + NPS)
        plsc.addupdate_scatter(deg_v, [dv - base], ones, mask=keep)

    for i in range(NPS // L):
        d = deg_v[pl.ds(i * L, L)]
        y = plsc.bitcast(
            jnp.int32(0x5F3759DF) - (plsc.bitcast(d, jnp.int32) >> 1),
            jnp.float32)
        for _ in range(3):
            y = y * (1.5 - 0.5 * d * y * y)
        dis_v[pl.ds(i * L, L)] = y

    bias_vec = bias_v[...]

    def mm_body(t, carry):
        n0 = t * 4
        accs = [[jnp.zeros((L,), jnp.float32) for _ in range(4)]
                for _ in range(4)]
        for kb in range(IN_D // L):
            xvs = [x_v[n0 + u, pl.ds(kb * L, L)] for u in range(4)]
            for j in range(L):
                wrow = wt_v[kb * L + j, :]
                for u in range(4):
                    b = j % 4
                    accs[u][b] = accs[u][b] + xvs[u][j] * wrow
        for u in range(4):
            n = n0 + u
            a = accs[u]
            h8_v[n >> 3, pl.ds((n & 7) * L, L)] = (a[0] + a[1]) + (a[2] + a[3])
        return carry

    lax.fori_loop(0, NPS // 4, mm_body, 0)

    def scale_body(t, carry):
        n0 = t * L
        dvec = dis_v[pl.ds(n0, L)]
        dsq = dvec * dvec
        for u in range(L):
            n = n0 + u
            hrow = h8_v[n >> 3, pl.ds((n & 7) * L, L)]
            o_v[n, :] = hrow * dsq[u] + bias_vec
        return carry

    lax.fori_loop(0, NPS // L, scale_body, 0)

    pltpu.sync_copy(h8_v, h_hbm.at[pl.ds(s * (NPS // 8), NPS // 8), :])
    pltpu.sync_copy(dis_v, dis_hbm.at[pl.ds(base, NPS)])
    plsc.subcore_barrier()

    pltpu.sync_copy(dis_hbm, disall_v)
    pltpu.sync_copy(h_hbm, hall_v)

    def edge_body(g, carry):
        e0 = g * L
        sv = src_v[pl.ds(e0, L)]
        dv = dst_v[pl.ds(e0, L)]
        ew = jnp.where(sv != dv, 1.0, 0.0).astype(jnp.float32)
        dis_s = plsc.load_gather(disall_v, [sv])
        dis_d = plsc.load_gather(disall_v, [dv])
        nv = dis_s * ew * dis_d
        ldv = dv - base
        for j in range(L):
            lj = ldv[j]
            nj = nv[j]
            sj = sv[j]

            @pl.when((lj >= 0) & (lj < NPS))
            def _():
                hrow = hall_v[sj >> 3, pl.ds((sj & 7) * L, L)]
                o_v[lj, :] = o_v[lj, :] + hrow * nj

        return carry

    lax.fori_loop(0, E_PAD // L, edge_body, 0)

    pltpu.sync_copy(o_v, out_hbm.at[pl.ds(base, NPS), :])


@jax.jit
def _run(x_pad, wt, bias, src_pad, dst_pad):
    mesh = plsc.VectorSubcoreMesh(
        core_axis_name="c", subcore_axis_name="s", num_cores=1,
        num_subcores=16)
    f = pl.kernel(
        _sc_body,
        out_type=(jax.ShapeDtypeStruct((N_PAD, OUT_D), jnp.float32),
                  jax.ShapeDtypeStruct((N_PAD // 8, 8 * OUT_D), jnp.float32),
                  jax.ShapeDtypeStruct((N_PAD,), jnp.float32)),
        mesh=mesh,
        scratch_types=[
            pltpu.VMEM((NPS, IN_D), jnp.float32),
            pltpu.VMEM((IN_D, OUT_D), jnp.float32),
            pltpu.VMEM((OUT_D,), jnp.float32),
            pltpu.VMEM((E_PAD,), jnp.int32),
            pltpu.VMEM((E_PAD,), jnp.int32),
            pltpu.VMEM((NPS,), jnp.float32),
            pltpu.VMEM((NPS,), jnp.float32),
            pltpu.VMEM((N_PAD,), jnp.float32),
            pltpu.VMEM((NPS // 8, 8 * OUT_D), jnp.float32),
            pltpu.VMEM((NPS, OUT_D), jnp.float32),
            pltpu.VMEM((N_PAD // 8, 8 * OUT_D), jnp.float32),
        ],
        compiler_params=pltpu.CompilerParams(needs_layout_passes=False),
    )
    return f(x_pad, wt, bias, src_pad, dst_pad)


def kernel(L_args_0_, L_args_1_,
           L_self_modules_encoder_modules_lin_parameters_weight_,
           L_self_modules_encoder_parameters_bias_):
    x = L_args_0_
    edge_index = L_args_1_.astype(jnp.int32)
    weight = L_self_modules_encoder_modules_lin_parameters_weight_
    bias = L_self_modules_encoder_parameters_bias_

    n_nodes, _ = x.shape
    n_edges = edge_index.shape[1]
    x_pad = jnp.pad(x, ((0, N_PAD - n_nodes), (0, 0)))
    src_pad = jnp.pad(edge_index[0], (0, E_PAD - n_edges))
    dst_pad = jnp.pad(edge_index[1], (0, E_PAD - n_edges))
    wt = weight.T

    out, _, _ = _run(x_pad, wt, bias, src_pad, dst_pad)
    return (out[:n_nodes],)


# speedup vs baseline: 2.0982x; 1.0008x over previous
import jax
import jax.numpy as jnp
from jax import lax
from jax.experimental import pallas as pl
from jax.experimental.pallas import tpu as pltpu
from jax.experimental.pallas import tpu_sc as plsc

N_NODES = 1000
N_PAD = 1024
NPS = 64
LAST = N_NODES - 15 * NPS
E_PAD = 112
IN_D = 128
OUT_D = 16
L = 16


def _sc_body(x_hbm, wt_hbm, bias_hbm, src_hbm, dst_hbm,
             out_hbm, h_hbm, dis_hbm,
             x_v, wt_v, bias_v, src_v, dst_v,
             deg_v, dis_v, disall_v, h8_v, o_v, hall_v):
    s = lax.axis_index("s")
    base = s * NPS

    @pl.when(s < 15)
    def _():
        pltpu.sync_copy(x_hbm.at[pl.ds(base, NPS), :], x_v)

    @pl.when(s == 15)
    def _():
        pltpu.sync_copy(x_hbm.at[pl.ds(15 * NPS, LAST), :],
                        x_v.at[pl.ds(0, LAST), :])

    pltpu.sync_copy(wt_hbm, wt_v)
    pltpu.sync_copy(bias_hbm, bias_v)
    pltpu.sync_copy(src_hbm, src_v)
    pltpu.sync_copy(dst_hbm, dst_v)

    ones = jnp.full((L,), 1.0, jnp.float32)
    for i in range(NPS // L):
        deg_v[pl.ds(i * L, L)] = ones

    for g in range(E_PAD // L):
        sv = src_v[pl.ds(g * L, L)]
        dv = dst_v[pl.ds(g * L, L)]
        keep = (sv != dv) & (dv >= base) & (dv < base + NPS)
        plsc.addupdate_scatter(deg_v, [dv - base], ones, mask=keep)

    for i in range(NPS // L):
        d = deg_v[pl.ds(i * L, L)]
        y = plsc.bitcast(
            jnp.int32(0x5F3759DF) - (plsc.bitcast(d, jnp.int32) >> 1),
            jnp.float32)
        for _ in range(3):
            y = y * (1.5 - 0.5 * d * y * y)
        dis_v[pl.ds(i * L, L)] = y

    bias_vec = bias_v[...]

    def mm_body(t, carry):
        n0 = t * 4
        accs = [[jnp.zeros((L,), jnp.float32) for _ in range(4)]
                for _ in range(4)]
        for kb in range(IN_D // L):
            xvs = [x_v[n0 + u, pl.ds(kb * L, L)] for u in range(4)]
            for j in range(L):
                wrow = wt_v[kb * L + j, :]
                for u in range(4):
                    b = j % 4
                    accs[u][b] = accs[u][b] + xvs[u][j] * wrow
        for u in range(4):
            n = n0 + u
            a = accs[u]
            h8_v[n >> 3, pl.ds((n & 7) * L, L)] = (a[0] + a[1]) + (a[2] + a[3])
        return carry

    lax.fori_loop(0, NPS // 4, mm_body, 0)

    def scale_body(t, carry):
        n0 = t * L
        dvec = dis_v[pl.ds(n0, L)]
        dsq = dvec * dvec
        for u in range(L):
            n = n0 + u
            hrow = h8_v[n >> 3, pl.ds((n & 7) * L, L)]
            o_v[n, :] = hrow * dsq[u] + bias_vec
        return carry

    lax.fori_loop(0, NPS // L, scale_body, 0)

    pltpu.sync_copy(h8_v, h_hbm.at[pl.ds(s * (NPS // 8), NPS // 8), :])
    pltpu.sync_copy(dis_v, dis_hbm.at[pl.ds(base, NPS)])
    plsc.subcore_barrier()

    pltpu.sync_copy(dis_hbm, disall_v)
    pltpu.sync_copy(h_hbm, hall_v)

    def edge_body(g, carry):
        e0 = g * L
        sv = src_v[pl.ds(e0, L)]
        dv = dst_v[pl.ds(e0, L)]
        ew = jnp.where(sv != dv, 1.0, 0.0).astype(jnp.float32)
        dis_s = plsc.load_gather(disall_v, [sv])
        dis_d = plsc.load_gather(disall_v, [dv])
        nv = dis_s * ew * dis_d
        ldv = dv - base
        for j in range(L):
            lj = ldv[j]
            nj = nv[j]
            sj = sv[j]

            @pl.when((lj >= 0) & (lj < NPS))
            def _():
                hrow = hall_v[sj >> 3, pl.ds((sj & 7) * L, L)]
                o_v[lj, :] = o_v[lj, :] + hrow * nj

        return carry

    lax.fori_loop(0, E_PAD // L, edge_body, 0)

    @pl.when(s < 15)
    def _():
        pltpu.sync_copy(o_v, out_hbm.at[pl.ds(base, NPS), :])

    @pl.when(s == 15)
    def _():
        pltpu.sync_copy(o_v.at[pl.ds(0, LAST), :],
                        out_hbm.at[pl.ds(15 * NPS, LAST), :])


@jax.jit
def _run(x, wt, bias, src_pad, dst_pad):
    mesh = plsc.VectorSubcoreMesh(
        core_axis_name="c", subcore_axis_name="s", num_cores=1,
        num_subcores=16)
    f = pl.kernel(
        _sc_body,
        out_type=(jax.ShapeDtypeStruct((N_NODES, OUT_D), jnp.float32),
                  jax.ShapeDtypeStruct((N_PAD // 8, 8 * OUT_D), jnp.float32),
                  jax.ShapeDtypeStruct((N_PAD,), jnp.float32)),
        mesh=mesh,
        scratch_types=[
            pltpu.VMEM((NPS, IN_D), jnp.float32),
            pltpu.VMEM((IN_D, OUT_D), jnp.float32),
            pltpu.VMEM((OUT_D,), jnp.float32),
            pltpu.VMEM((E_PAD,), jnp.int32),
            pltpu.VMEM((E_PAD,), jnp.int32),
            pltpu.VMEM((NPS,), jnp.float32),
            pltpu.VMEM((NPS,), jnp.float32),
            pltpu.VMEM((N_PAD,), jnp.float32),
            pltpu.VMEM((NPS // 8, 8 * OUT_D), jnp.float32),
            pltpu.VMEM((NPS, OUT_D), jnp.float32),
            pltpu.VMEM((N_PAD // 8, 8 * OUT_D), jnp.float32),
        ],
        compiler_params=pltpu.CompilerParams(needs_layout_passes=False),
    )
    return f(x, wt, bias, src_pad, dst_pad)


def kernel(L_args_0_, L_args_1_,
           L_self_modules_encoder_modules_lin_parameters_weight_,
           L_self_modules_encoder_parameters_bias_):
    x = L_args_0_
    edge_index = L_args_1_.astype(jnp.int32)
    weight = L_self_modules_encoder_modules_lin_parameters_weight_
    bias = L_self_modules_encoder_parameters_bias_

    n_edges = edge_index.shape[1]
    src_pad = jnp.pad(edge_index[0], (0, E_PAD - n_edges))
    dst_pad = jnp.pad(edge_index[1], (0, E_PAD - n_edges))
    wt = weight.T

    out, _, _ = _run(x, wt, bias, src_pad, dst_pad)
    return (out,)
